# baseline (speedup 1.0000x reference)
import functools

import jax
import jax.numpy as jnp
from jax.experimental import pallas as pl
from jax.experimental.pallas import tpu as pltpu

_NEG = 0.01
_L = 2
_W_BUY, _W_CART, _W_PV = 0.5, 0.25, 0.25


def _mm_scale_leaky_body(x_ref, w_ref, s_ref, o_ref):
    y = jnp.dot(x_ref[...], w_ref[...], preferred_element_type=jnp.float32)
    y = y * s_ref[...]
    o_ref[...] = jnp.where(y >= 0, y, _NEG * y)


def _mm_leaky_body(x_ref, w_ref, o_ref):
    y = jnp.dot(x_ref[...], w_ref[...], preferred_element_type=jnp.float32)
    o_ref[...] = jnp.where(y >= 0, y, _NEG * y)


def _mm_scale_leaky(x, w, scale, blk=1000):
    n = x.shape[0]
    return pl.pallas_call(
        _mm_scale_leaky_body,
        grid=(n // blk,),
        in_specs=[
            pl.BlockSpec((blk, 128), lambda i: (i, 0)),
            pl.BlockSpec((128, 128), lambda i: (0, 0)),
            pl.BlockSpec((blk, 1), lambda i: (i, 0)),
        ],
        out_specs=pl.BlockSpec((blk, 128), lambda i: (i, 0)),
        out_shape=jax.ShapeDtypeStruct((n, 128), jnp.float32),
    )(x, w, scale.reshape(n, 1))


def _mm_leaky(x, w, blk=1000):
    n = x.shape[0]
    return pl.pallas_call(
        _mm_leaky_body,
        grid=(n // blk,),
        in_specs=[
            pl.BlockSpec((blk, 128), lambda i: (i, 0)),
            pl.BlockSpec((128, 128), lambda i: (0, 0)),
        ],
        out_specs=pl.BlockSpec((blk, 128), lambda i: (i, 0)),
        out_shape=jax.ShapeDtypeStruct((n, 128), jnp.float32),
    )(x, w)


def kernel(buy_edges, cart_edges, pv_edges, user_emb, item_emb,
           buy_edges_emb, cart_edges_emb, pv_edges_emb, node_w, edge_w):
    a = 0.0045
    b = 0.0045
    nu = user_emb.shape[0]
    ni = item_emb.shape[0]

    rels = [
        (buy_edges[0], buy_edges[1], buy_edges_emb * b, _W_BUY),
        (cart_edges[0], cart_edges[1], cart_edges_emb * b, _W_CART),
        (pv_edges[0], pv_edges[1], pv_edges_emb * b, _W_PV),
    ]

    pre = []
    for (u, v, _, w) in rels:
        du = jnp.clip(jnp.zeros((nu,), jnp.float32).at[u].add(1.0), 1.0)
        dv = jnp.clip(jnp.zeros((ni,), jnp.float32).at[v].add(1.0), 1.0)
        oinv = du ** -0.5
        iinv = dv ** -0.5
        cu = oinv[u]
        cv = iinv[v]
        invden = 1.0 / (du[u] + dv[v])
        pre.append((u, v, w, oinv, iinv, cu, cv, invden))

    src = user_emb * a
    dst = item_emb * a
    es = [r[2] for r in rels]
    src_all = src
    dst_all = dst

    for i in range(_L):
        W = node_w[i]
        We = edge_w[i]
        new_dst = jnp.zeros((ni, 128), jnp.float32)
        new_src = jnp.zeros((nu, 128), jnp.float32)
        new_es = []
        for (u, v, w, oinv, iinv, cu, cv, invden), e in zip(pre, es):
            m1 = src[u] * (cu[:, None] * e)
            agg_v = jnp.zeros((ni, 128), jnp.float32).at[v].add(m1)
            m2 = dst[v] * (cv[:, None] * e)
            agg_u = jnp.zeros((nu, 128), jnp.float32).at[u].add(m2)
            a_u = jnp.zeros((nu, 128), jnp.float32).at[u].add(e)
            a_v = jnp.zeros((ni, 128), jnp.float32).at[v].add(e)
            t = (a_u[u] + a_v[v]) * invden[:, None]
            new_dst = new_dst + _mm_scale_leaky(agg_v, W, w * iinv)
            new_src = new_src + _mm_scale_leaky(agg_u, W, w * oinv)
            new_es.append(_mm_leaky(t, We))
        src, dst, es = new_src, new_dst, new_es
        src_all = src_all + src
        dst_all = dst_all + dst

    return (src_all / (_L + 1), dst_all / (_L + 1))
